# baseline (speedup 1.0000x reference)
import functools

import jax
import jax.numpy as jnp
from jax import lax
from jax.experimental import pallas as pl
from jax.experimental.pallas import tpu as pltpu
from jax.experimental.pallas import tpu_sc as plsc

_NC, _NS = 2, 16
_NW = _NC * _NS


def _make_gather(rows, n, h):
    b_per_w = rows // _NW
    mesh = plsc.VectorSubcoreMesh(core_axis_name="c", subcore_axis_name="s")

    @functools.partial(
        pl.kernel,
        mesh=mesh,
        out_type=jax.ShapeDtypeStruct((rows, h), jnp.float32),
        scratch_types=[
            pltpu.VMEM((b_per_w,), jnp.int32),
            pltpu.VMEM((b_per_w, h), jnp.float32),
            pltpu.SemaphoreType.DMA,
        ],
    )
    def gather_rows(table_hbm, idx_hbm, out_hbm, idx_v, rows_v, sem):
        wid = lax.axis_index("s") * _NC + lax.axis_index("c")
        base = wid * b_per_w
        pltpu.sync_copy(idx_hbm.at[pl.ds(base, b_per_w)], idx_v)
        flat = idx_v[...] + (base + lax.iota(jnp.int32, b_per_w)) * n
        idx_v[...] = flat
        pltpu.async_copy(table_hbm.at[idx_v], rows_v, sem).wait()
        pltpu.sync_copy(rows_v, out_hbm.at[pl.ds(base, b_per_w)])

    return gather_rows


def _mlp_body(pair_ref, w1_ref, b1_ref, w2_ref, b2_ref, out_ref, *, out_dim):
    hidden = jnp.dot(pair_ref[...], w1_ref[...],
                     preferred_element_type=jnp.float32)
    hidden = jnp.maximum(hidden + b1_ref[...], 0.0)
    logits = jnp.dot(hidden, w2_ref[...],
                     preferred_element_type=jnp.float32) + b2_ref[...]
    valid = lax.broadcasted_iota(jnp.int32, logits.shape, 1) < out_dim
    masked = jnp.where(valid, logits, -jnp.inf)
    m = jnp.max(masked, axis=1, keepdims=True)
    e = jnp.where(valid, jnp.exp(masked - m), 0.0)
    out_ref[...] = e / jnp.sum(e, axis=1, keepdims=True)


def kernel(directed_graph_encodings, previous_ids, W1, b1, W2, b2):
    b, n, _, h = directed_graph_encodings.shape
    out_dim = W2.shape[1]
    rows = b * n
    table = directed_graph_encodings.reshape(rows * n, h)
    idx = previous_ids.reshape(rows).astype(jnp.int32)
    pair = _make_gather(rows, n, h)(table, idx)
    w2p = jnp.pad(W2, ((0, 0), (0, 128 - out_dim)))
    b2p = jnp.pad(b2, (0, 128 - out_dim)).reshape(1, 128)
    probs = pl.pallas_call(
        functools.partial(_mlp_body, out_dim=out_dim),
        out_shape=jax.ShapeDtypeStruct((rows, 128), jnp.float32),
    )(pair, W1, b1.reshape(1, h), w2p, b2p)
    return probs.reshape(b, n, 128)[:, 1:, :out_dim]
